# baseline (speedup 1.0000x reference)
import functools

import jax
import jax.numpy as jnp
from jax.experimental import pallas as pl
from jax.experimental.pallas import tpu as pltpu

_BN = 256


def _fused_gc_body(d_ref, Tmf_ref, Tmb_ref, adj_ref, HW_ref, b_ref, out_ref,
                   *, bm, relu):
    i = pl.program_id(0)
    Tsc = (Tmf_ref[...] * d_ref[...]).astype(jnp.bfloat16)
    mult = jax.lax.dot_general(
        Tsc, Tmb_ref[...], (((1,), (1,)), ((), ())),
        preferred_element_type=jnp.float32,
    )
    adj = adj_ref[...]
    R = adj.shape[1]
    rows = jax.lax.broadcasted_iota(jnp.int32, (bm, R), 0) + i * bm
    cols = jax.lax.broadcasted_iota(jnp.int32, (bm, R), 1)
    A = jnp.where(rows == cols, adj, mult * adj).astype(jnp.bfloat16)
    out = jax.lax.dot_general(
        A, HW_ref[...], (((1,), (0,)), ((), ())),
        preferred_element_type=jnp.float32,
    ) + b_ref[...]
    if relu:
        out = jnp.maximum(out, 0.0)
    out_ref[...] = out


def _fused_gc(Tmf, Tmb, d, adj, HW, b, relu):
    R, K = Tmf.shape
    F = HW.shape[1]
    bm = _BN
    num_i = R // bm
    return pl.pallas_call(
        functools.partial(_fused_gc_body, bm=bm, relu=relu),
        grid=(num_i,),
        in_specs=[
            pl.BlockSpec((1, K), lambda i: (0, 0)),
            pl.BlockSpec((bm, K), lambda i: (i, 0)),
            pl.BlockSpec((R, K), lambda i: (0, 0)),
            pl.BlockSpec((bm, R), lambda i: (i, 0)),
            pl.BlockSpec((R, F), lambda i: (0, 0)),
            pl.BlockSpec((1, F), lambda i: (0, 0)),
        ],
        out_specs=pl.BlockSpec((bm, F), lambda i: (i, 0)),
        out_shape=jax.ShapeDtypeStruct((R, F), jnp.float32),
        compiler_params=pltpu.CompilerParams(
            vmem_limit_bytes=56 * 1024 * 1024,
        ),
    )(d.reshape(1, K), Tmf, Tmb, adj, HW.astype(jnp.bfloat16),
      b.reshape(1, F))


def _layernorm(h, g, be):
    m = jnp.mean(h, axis=-1, keepdims=True)
    v = jnp.mean((h - m) ** 2, axis=-1, keepdims=True)
    return (h - m) / jnp.sqrt(v + 1e-5) * g + be


def _pad_p(p):
    K = p.shape[1]
    return jnp.zeros((K, 128), jnp.float32).at[:, 0].set(p[0])


def _d_dot(H, Ppad_ref):
    return jax.lax.dot_general(
        H, Ppad_ref[...], (((1,), (0,)), ((), ())),
        preferred_element_type=jnp.float32)[:, :1]


def _prep1_body(X_ref, Z_ref, W_ref, p_ref, fW_ref, g_ref, be_ref,
                d_ref, HW_ref, F1_ref):
    X = X_ref[...]
    Z = Z_ref[...]
    d_ref[...] = _d_dot(Z, p_ref)
    HW_ref[...] = jnp.dot(X, W_ref[...], preferred_element_type=jnp.float32)
    h = jnp.dot(X, fW_ref[...], preferred_element_type=jnp.float32)
    F1_ref[...] = jnp.maximum(_layernorm(h, g_ref[...], be_ref[...]), 0.0)


def _prep1(X, Z, W, p, fW, g, be):
    N_, NFV_ = X.shape
    E_ = Z.shape[0]
    NH = W.shape[1]
    return pl.pallas_call(
        _prep1_body,
        out_shape=(
            jax.ShapeDtypeStruct((E_, 1), jnp.float32),
            jax.ShapeDtypeStruct((N_, NH), jnp.float32),
            jax.ShapeDtypeStruct((N_, NH), jnp.float32),
        ),
    )(X, Z, W, _pad_p(p), fW, g.reshape(1, NH), be.reshape(1, NH))


def _prep2_body(X1F1_ref, Z_ref, W_ref, p_ref, fW_ref, g_ref, be_ref,
                d_ref, HeW_ref, F2_ref):
    Z = Z_ref[...]
    d_ref[...] = _d_dot(X1F1_ref[...], p_ref)
    Z1 = jnp.maximum(Z, 0.0)
    HeW_ref[...] = jnp.dot(Z1, W_ref[...], preferred_element_type=jnp.float32)
    h = jnp.dot(Z, fW_ref[...], preferred_element_type=jnp.float32)
    F2_ref[...] = jnp.maximum(_layernorm(h, g_ref[...], be_ref[...]), 0.0)


def _prep2(X1F1, Z, W, p, fW, g, be):
    N_ = X1F1.shape[0]
    E_, NFE_ = Z.shape
    return pl.pallas_call(
        _prep2_body,
        out_shape=(
            jax.ShapeDtypeStruct((N_, 1), jnp.float32),
            jax.ShapeDtypeStruct((E_, NFE_), jnp.float32),
            jax.ShapeDtypeStruct((E_, NFE_), jnp.float32),
        ),
    )(X1F1, Z, W, _pad_p(p), fW, g.reshape(1, NFE_), be.reshape(1, NFE_))


def _prep35_body(Hv_ref, He_ref, W_ref, p_ref, d_ref, HW_ref):
    d_ref[...] = _d_dot(He_ref[...], p_ref)
    HW_ref[...] = jnp.dot(Hv_ref[...], W_ref[...],
                          preferred_element_type=jnp.float32)


def _prep35(Hv, He, W, p):
    return pl.pallas_call(
        _prep35_body,
        out_shape=(
            jax.ShapeDtypeStruct((He.shape[0], 1), jnp.float32),
            jax.ShapeDtypeStruct((Hv.shape[0], W.shape[1]), jnp.float32),
        ),
    )(Hv, He, W, _pad_p(p))


def _prep4_body(Hv_ref, He_ref, W_ref, p_ref, d_ref, HeW_ref):
    d_ref[...] = _d_dot(Hv_ref[...], p_ref)
    HeW_ref[...] = jnp.dot(He_ref[...], W_ref[...],
                           preferred_element_type=jnp.float32)


def _prep4(Hv, He, W, p):
    return pl.pallas_call(
        _prep4_body,
        out_shape=(
            jax.ShapeDtypeStruct((Hv.shape[0], 1), jnp.float32),
            jax.ShapeDtypeStruct((He.shape[0], W.shape[1]), jnp.float32),
        ),
    )(Hv, He, W, _pad_p(p))


def kernel(X, Z, adj_e, adj_v, T, gc1_W, gc1_p, gc1_b, fc1_W, fc1_g, fc1_be,
           gc2_W, gc2_p, gc2_b, fc2_W, fc2_g, fc2_be, gc3_W, gc3_p, gc3_b,
           gc4_W, gc4_p, gc4_b, gc5_W, gc5_p, gc5_b):
    Tb = T.astype(jnp.bfloat16)
    Tt, Ttb = jax.lax.optimization_barrier((T.T, Tb.T))

    d1, HW1, F1 = _prep1(X, Z, gc1_W, gc1_p, fc1_W, fc1_g, fc1_be)
    X1 = _fused_gc(T, Tb, d1, adj_v, HW1, gc1_b, relu=True)
    X1F1 = jnp.concatenate([X1, F1], axis=1)

    d2, HeW2, F2 = _prep2(X1F1, Z, gc2_W, gc2_p, fc2_W, fc2_g, fc2_be)
    Z2 = _fused_gc(Tt, Ttb, d2, adj_e, HeW2, gc2_b, relu=True)
    Z2F2 = jnp.concatenate([Z2, F2], axis=1)

    d3, HW3 = _prep35(X1F1, Z2F2, gc3_W, gc3_p)
    X3 = _fused_gc(T, Tb, d3, adj_v, HW3, gc3_b, relu=True)

    d4, HeW4 = _prep4(X3, Z2F2, gc4_W, gc4_p)
    Z4 = _fused_gc(Tt, Ttb, d4, adj_e, HeW4, gc4_b, relu=True)

    d5, HW5 = _prep35(X3, Z4, gc5_W, gc5_p)
    X5 = _fused_gc(T, Tb, d5, adj_v, HW5, gc5_b, relu=False)
    return X5
